# baseline (speedup 1.0000x reference)
import functools
import math

import jax
import jax.numpy as jnp
from jax.experimental import pallas as pl
from jax.experimental.pallas import tpu as pltpu


def _router_kernel(x_ref, w_ref, b_ref, idx_ref, score_ref, acc_ref,
                   counts_ref, *, blk, n_expert, capacity, nk):
    i = pl.program_id(0)
    k = pl.program_id(1)

    @pl.when(jnp.logical_and(i == 0, k == 0))
    def _init():
        counts_ref[...] = jnp.zeros_like(counts_ref)

    @pl.when(k == 0)
    def _zero_acc():
        acc_ref[...] = jnp.zeros_like(acc_ref)

    acc_ref[...] += jax.lax.dot_general(
        x_ref[...], w_ref[...], dimension_numbers=(((1,), (1,)), ((), ())),
        preferred_element_type=jnp.float32)

    @pl.when(k == nk - 1)
    def _route():
        logits = acc_ref[...] + b_ref[...]

        m = jnp.max(logits, axis=1, keepdims=True)
        denom = jnp.sum(jnp.exp(logits - m), axis=1)
        score = 1.0 / denom

        lane = jax.lax.broadcasted_iota(jnp.int32, (blk, n_expert), 1)
        idx = jnp.min(jnp.where(logits == m, lane, n_expert), axis=1)
        onehot = (lane == idx[:, None]).astype(jnp.float32)

        ri = jax.lax.broadcasted_iota(jnp.int32, (blk, blk), 0)
        ci = jax.lax.broadcasted_iota(jnp.int32, (blk, blk), 1)
        tri = (ri >= ci).astype(jnp.float32)
        cs = jax.lax.dot_general(
            tri, onehot, dimension_numbers=(((1,), (0,)), ((), ())),
            preferred_element_type=jnp.float32)

        prev = counts_ref[...]
        pos = jnp.sum((cs + prev) * onehot, axis=1) - 1.0
        pruned = jnp.where(pos < capacity, idx, -1)

        counts_ref[...] = prev + jnp.sum(onehot, axis=0, keepdims=True)

        idx_ref[...] = pruned[:, None].astype(jnp.int32)
        score_ref[...] = score[:, None]


@jax.jit
def kernel(inp, W, b):
    n, d = inp.shape
    e = W.shape[0]
    blk = 1024
    dk = 1024
    nk = d // dk
    capacity = math.ceil(2.4 * n / e)

    idx_out, score_out = pl.pallas_call(
        functools.partial(_router_kernel, blk=blk, n_expert=e,
                          capacity=capacity, nk=nk),
        grid=(n // blk, nk),
        in_specs=[
            pl.BlockSpec((blk, dk), lambda i, k: (i, k)),
            pl.BlockSpec((e, dk), lambda i, k: (0, k)),
            pl.BlockSpec((1, e), lambda i, k: (0, 0)),
        ],
        out_specs=[
            pl.BlockSpec((blk, 1), lambda i, k: (i, 0)),
            pl.BlockSpec((blk, 1), lambda i, k: (i, 0)),
        ],
        out_shape=[
            jax.ShapeDtypeStruct((n, 1), jnp.int32),
            jax.ShapeDtypeStruct((n, 1), jnp.float32),
        ],
        scratch_shapes=[pltpu.VMEM((blk, e), jnp.float32),
                        pltpu.VMEM((1, e), jnp.float32)],
    )(inp, W, b.reshape(1, e))
    return (idx_out, score_out)


# speedup vs baseline: 1.2734x; 1.2734x over previous
import functools
import math

import jax
import jax.numpy as jnp
from jax.experimental import pallas as pl
from jax.experimental.pallas import tpu as pltpu


def _router_kernel(x_ref, w_ref, b_ref, idx_ref, score_ref, counts_ref,
                   *, blk, n_expert, capacity):
    step = pl.program_id(0)

    @pl.when(step == 0)
    def _init():
        counts_ref[...] = jnp.zeros_like(counts_ref)

    x = x_ref[...]
    w = w_ref[...]
    logits = jax.lax.dot_general(
        x, w, dimension_numbers=(((1,), (1,)), ((), ())),
        preferred_element_type=jnp.float32)
    logits = logits + b_ref[...]

    m = jnp.max(logits, axis=1, keepdims=True)
    denom = jnp.sum(jnp.exp(logits - m), axis=1)
    score = 1.0 / denom

    lane = jax.lax.broadcasted_iota(jnp.int32, (blk, n_expert), 1)
    idx = jnp.min(jnp.where(logits == m, lane, n_expert), axis=1)
    onehot = (lane == idx[:, None]).astype(jnp.float32)

    ri = jax.lax.broadcasted_iota(jnp.int32, (blk, blk), 0)
    ci = jax.lax.broadcasted_iota(jnp.int32, (blk, blk), 1)
    tri = (ri >= ci).astype(jnp.float32)
    cs = jax.lax.dot_general(
        tri, onehot, dimension_numbers=(((1,), (0,)), ((), ())),
        preferred_element_type=jnp.float32)

    prev = counts_ref[...]
    pos = jnp.sum((cs + prev) * onehot, axis=1) - 1.0
    pruned = jnp.where(pos < capacity, idx, -1)

    counts_ref[...] = prev + jnp.sum(onehot, axis=0, keepdims=True)

    idx_ref[...] = pruned[:, None].astype(jnp.int32)
    score_ref[...] = score[:, None]


@jax.jit
def kernel(inp, W, b):
    n, d = inp.shape
    e = W.shape[0]
    blk = 1024
    capacity = math.ceil(2.4 * n / e)
    grid = n // blk

    idx_out, score_out = pl.pallas_call(
        functools.partial(_router_kernel, blk=blk, n_expert=e,
                          capacity=capacity),
        grid=(grid,),
        in_specs=[
            pl.BlockSpec((blk, d), lambda i: (i, 0)),
            pl.BlockSpec((e, d), lambda i: (0, 0)),
            pl.BlockSpec((1, e), lambda i: (0, 0)),
        ],
        out_specs=[
            pl.BlockSpec((blk, 1), lambda i: (i, 0)),
            pl.BlockSpec((blk, 1), lambda i: (i, 0)),
        ],
        out_shape=[
            jax.ShapeDtypeStruct((n, 1), jnp.int32),
            jax.ShapeDtypeStruct((n, 1), jnp.float32),
        ],
        scratch_shapes=[pltpu.VMEM((1, e), jnp.float32)],
    )(inp, W, b.reshape(1, e))
    return (idx_out, score_out)
